# baseline (speedup 1.0000x reference)
import functools

import jax
import jax.numpy as jnp
from jax import lax
from jax.experimental import pallas as pl
from jax.experimental.pallas import tpu as pltpu
from jax.experimental.pallas import tpu_sc as plsc

VOCAB = 1000000
D = 32
B = 4096
L = 200

NC, NS = 2, 16
NW = NC * NS
BPW = B // NW
NG = BPW // 16
FT, FS = D // 8, 8

_mesh = plsc.VectorSubcoreMesh(core_axis_name="c", subcore_axis_name="s")


@functools.partial(
    pl.kernel,
    mesh=_mesh,
    out_type=jax.ShapeDtypeStruct((L, FT, NW, FS, BPW), jnp.float32),
    compiler_params=pltpu.CompilerParams(
        use_tc_tiling_on_sc=False, needs_layout_passes=False),
    scratch_types=[
        pltpu.VMEM((L, BPW), jnp.int32),
        pltpu.VMEM((L, D), jnp.float32),
        pltpu.VMEM((BPW, D), jnp.float32),
        pltpu.VMEM((BPW, D), jnp.float32),
        pltpu.VMEM((FT, FS, BPW), jnp.float32),
        pltpu.VMEM((FT, FS, BPW), jnp.float32),
        pltpu.SemaphoreType.DMA,
        pltpu.SemaphoreType.DMA,
        pltpu.SemaphoreType.DMA,
        pltpu.SemaphoreType.DMA,
    ],
)
def _embed_sc(table_hbm, idx_hbm, pe_hbm, out_hbm,
              idx_v, pe_v, rows0, rows1, ob0, ob1,
              sg0, sg1, so0, so1):
    wid = lax.axis_index("s") * NC + lax.axis_index("c")
    b0 = wid * BPW

    pltpu.sync_copy(idx_hbm.at[:, pl.ds(b0, BPW)], idx_v)
    pltpu.sync_copy(pe_hbm, pe_v)

    rows = (rows0, rows1)
    ob = (ob0, ob1)
    sg = (sg0, sg1)
    so = (so0, so1)

    iota = lax.iota(jnp.int32, 16)
    krow = [iota + (16 * g) for g in range(NG)]

    def splat(v):
        return lax.broadcast(v, (16,))

    def stage(l, par):
        pltpu.async_copy(table_hbm.at[idx_v.at[l]], rows[par], sg[par])

    def emit(l, par):
        pltpu.make_async_copy(
            table_hbm.at[idx_v.at[l]], rows[par], sg[par]).wait()

        @pl.when(l >= 2)
        def _():
            pltpu.make_async_copy(
                ob[par], out_hbm.at[l, :, wid, :, :], so[par]).wait()

        for ft in range(FT):
            for fs in range(FS):
                f = ft * FS + fs
                ps = plsc.load_gather(pe_v, [splat(l), splat(jnp.int32(f))])
                for g in range(NG):
                    cv = plsc.load_gather(
                        rows[par], [krow[g], splat(jnp.int32(f))])
                    ob[par][ft, fs, pl.ds(16 * g, 16)] = cv + ps

        pltpu.async_copy(ob[par], out_hbm.at[l, :, wid, :, :], so[par])

    stage(0, 0)
    stage(1, 1)

    def pair_body(p, carry):
        l = p * 2
        emit(l, 0)

        @pl.when(l + 2 < L)
        def _():
            stage(l + 2, 0)
        emit(l + 1, 1)

        @pl.when(l + 3 < L)
        def _():
            stage(l + 3, 1)
        return carry

    lax.fori_loop(0, L // 2, pair_body, 0)
    pltpu.make_async_copy(
        ob0, out_hbm.at[L - 2, :, wid, :, :], so0).wait()
    pltpu.make_async_copy(
        ob1, out_hbm.at[L - 1, :, wid, :, :], so1).wait()


def kernel(x, table, pos_embedding):
    idx_t = jnp.transpose(x).astype(jnp.int32)
    pe = pos_embedding[:L, :].astype(jnp.float32)
    raw = _embed_sc(table, idx_t, pe)
    return jnp.transpose(raw, (2, 4, 0, 1, 3)).reshape(B, L, D)


# speedup vs baseline: 1.0901x; 1.0901x over previous
import functools

import jax
import jax.numpy as jnp
from jax import lax
from jax.experimental import pallas as pl
from jax.experimental.pallas import tpu as pltpu
from jax.experimental.pallas import tpu_sc as plsc

VOCAB = 1000000
D = 32
B = 4096
L = 200

NC, NS = 2, 16
NW = NC * NS
BPW = B // NW
NG = BPW // 16
FT, FS = D // 8, 8
CHL = 2
CHR = CHL * BPW
NCH = L // CHL
NBUF = 4
PER_W = L * BPW

_mesh = plsc.VectorSubcoreMesh(core_axis_name="c", subcore_axis_name="s")


@functools.partial(
    pl.kernel,
    mesh=_mesh,
    out_type=jax.ShapeDtypeStruct((L, FT, NW, FS, BPW), jnp.float32),
    compiler_params=pltpu.CompilerParams(
        use_tc_tiling_on_sc=False, needs_layout_passes=False),
    scratch_types=[
        pltpu.VMEM((PER_W,), jnp.int32),
        pltpu.VMEM((L, D), jnp.float32),
        pltpu.VMEM((CHR, D), jnp.float32),
        pltpu.VMEM((CHR, D), jnp.float32),
        pltpu.VMEM((CHR, D), jnp.float32),
        pltpu.VMEM((CHR, D), jnp.float32),
        pltpu.VMEM((CHL, FT, FS, BPW), jnp.float32),
        pltpu.VMEM((CHL, FT, FS, BPW), jnp.float32),
        pltpu.VMEM((CHL, FT, FS, BPW), jnp.float32),
        pltpu.VMEM((CHL, FT, FS, BPW), jnp.float32),
        pltpu.SemaphoreType.DMA,
        pltpu.SemaphoreType.DMA,
        pltpu.SemaphoreType.DMA,
        pltpu.SemaphoreType.DMA,
        pltpu.SemaphoreType.DMA,
        pltpu.SemaphoreType.DMA,
        pltpu.SemaphoreType.DMA,
        pltpu.SemaphoreType.DMA,
    ],
)
def _embed_sc(table_hbm, idx_hbm, pe_hbm, out_hbm,
              idx_v, pe_v, rows0, rows1, rows2, rows3,
              ob0, ob1, ob2, ob3,
              sg0, sg1, sg2, sg3, so0, so1, so2, so3):
    wid = lax.axis_index("s") * NC + lax.axis_index("c")

    pltpu.sync_copy(idx_hbm.at[pl.ds(wid * PER_W, PER_W)], idx_v)
    pltpu.sync_copy(pe_hbm, pe_v)

    rows = (rows0, rows1, rows2, rows3)
    ob = (ob0, ob1, ob2, ob3)
    sg = (sg0, sg1, sg2, sg3)
    so = (so0, so1, so2, so3)

    iota = lax.iota(jnp.int32, 16)

    def splat(v):
        return lax.broadcast(v, (16,))

    def start_gather(c, buf):
        pltpu.async_copy(
            table_hbm.at[idx_v.at[pl.ds(c * CHR, CHR)]], rows[buf], sg[buf])

    def wait_gather(c, buf):
        pltpu.make_async_copy(
            table_hbm.at[idx_v.at[pl.ds(c * CHR, CHR)]], rows[buf], sg[buf]
        ).wait()

    def start_write(c, buf):
        pltpu.async_copy(
            ob[buf], out_hbm.at[pl.ds(c * CHL, CHL), :, wid, :, :], so[buf])

    def wait_write(c, buf):
        pltpu.make_async_copy(
            ob[buf], out_hbm.at[pl.ds(c * CHL, CHL), :, wid, :, :], so[buf]
        ).wait()

    def transpose_emit(c, buf):
        for j in range(CHL):
            l = c * CHL + j

            def f_body(f, carry):
                ps = plsc.load_gather(pe_v, [splat(l), splat(f)])
                ft = f // FS
                fs = f % FS
                for g in range(NG):
                    kr = iota + (j * BPW + 16 * g)
                    cv = plsc.load_gather(rows[buf], [kr, splat(f)])
                    ob[buf][j, ft, fs, pl.ds(16 * g, 16)] = cv + ps
                return carry

            lax.fori_loop(0, D, f_body, 0)

    for b in range(NBUF):
        start_gather(b, b)

    def quad_body(p, carry):
        c0 = p * NBUF
        for b in range(NBUF):
            c = c0 + b
            wait_gather(c, b)
            transpose_emit(c, b)
            start_write(c, b)
            nxt = c + NBUF - 1
            pb = (b - 1) % NBUF

            @pl.when(jnp.logical_and(c >= 1, nxt < NCH))
            def _():
                wait_write(c - 1, pb)
                start_gather(nxt, pb)

        return carry

    lax.fori_loop(0, NCH // NBUF, quad_body, 0)

    for b in range(NBUF):
        c = NCH - NBUF + b
        wait_write(c, c % NBUF)


def kernel(x, table, pos_embedding):
    idx_w = (jnp.transpose(x).astype(jnp.int32)
             .reshape(L, NW, BPW).transpose(1, 0, 2).reshape(-1))
    pe = pos_embedding[:L, :].astype(jnp.float32)
    raw = _embed_sc(table, idx_w, pe)
    return jnp.transpose(raw, (2, 4, 0, 1, 3)).reshape(B, L, D)


# speedup vs baseline: 1.3372x; 1.2267x over previous
import functools

import jax
import jax.numpy as jnp
from jax import lax
from jax.experimental import pallas as pl
from jax.experimental.pallas import tpu as pltpu
from jax.experimental.pallas import tpu_sc as plsc

VOCAB = 1000000
D = 32
B = 4096
L = 200

NC, NS = 2, 16
NW = NC * NS
BPW = B // NW
NG = BPW // 16
FT, FS = D // 8, 8
CHL = 2
CHR = CHL * BPW
NCH = L // CHL
NBUF = 4
PER_W = L * BPW
PARTS = 2
PW = D // PARTS

_mesh = plsc.VectorSubcoreMesh(core_axis_name="c", subcore_axis_name="s")


@functools.partial(
    pl.kernel,
    mesh=_mesh,
    out_type=jax.ShapeDtypeStruct((L, FT, NW, FS, BPW), jnp.float32),
    compiler_params=pltpu.CompilerParams(
        use_tc_tiling_on_sc=False, needs_layout_passes=False),
    scratch_types=[
        pltpu.VMEM((PER_W,), jnp.int32),
        pltpu.VMEM((CHR, PW), jnp.float32),
        pltpu.VMEM((CHR, PW), jnp.float32),
        pltpu.VMEM((CHR, PW), jnp.float32),
        pltpu.VMEM((CHR, PW), jnp.float32),
        pltpu.VMEM((CHR, PW), jnp.float32),
        pltpu.VMEM((CHR, PW), jnp.float32),
        pltpu.VMEM((CHR, PW), jnp.float32),
        pltpu.VMEM((CHR, PW), jnp.float32),
        pltpu.VMEM((CHR,), jnp.int32),
        pltpu.VMEM((CHR,), jnp.int32),
        pltpu.VMEM((CHR,), jnp.int32),
        pltpu.VMEM((CHR,), jnp.int32),
        pltpu.VMEM((CHR,), jnp.int32),
        pltpu.VMEM((CHR,), jnp.int32),
        pltpu.VMEM((CHR,), jnp.int32),
        pltpu.VMEM((CHR,), jnp.int32),
        pltpu.VMEM((CHL, D, 16), jnp.float32),
        pltpu.VMEM((CHL, D, 16), jnp.float32),
        pltpu.VMEM((CHL, D, 16), jnp.float32),
        pltpu.VMEM((CHL, D, 16), jnp.float32),
        pltpu.VMEM((CHL, FT, FS, BPW), jnp.float32),
        pltpu.VMEM((CHL, FT, FS, BPW), jnp.float32),
        pltpu.VMEM((CHL, FT, FS, BPW), jnp.float32),
        pltpu.VMEM((CHL, FT, FS, BPW), jnp.float32),
        pltpu.SemaphoreType.DMA,
        pltpu.SemaphoreType.DMA,
        pltpu.SemaphoreType.DMA,
        pltpu.SemaphoreType.DMA,
        pltpu.SemaphoreType.DMA,
        pltpu.SemaphoreType.DMA,
        pltpu.SemaphoreType.DMA,
        pltpu.SemaphoreType.DMA,
        pltpu.SemaphoreType.DMA,
        pltpu.SemaphoreType.DMA,
        pltpu.SemaphoreType.DMA,
        pltpu.SemaphoreType.DMA,
    ],
)
def _embed_sc(table_hbm, idx_hbm, peb_hbm, out_hbm,
              idx_v,
              ra0, ra1, ra2, ra3, rb0, rb1, rb2, rb3,
              ga0, ga1, ga2, ga3, gb0, gb1, gb2, gb3,
              pb0, pb1, pb2, pb3,
              ob0, ob1, ob2, ob3,
              sa0, sa1, sa2, sa3, sb0, sb1, sb2, sb3,
              so0, so1, so2, so3):
    wid = lax.axis_index("s") * NC + lax.axis_index("c")

    pltpu.sync_copy(idx_hbm.at[pl.ds(wid * PER_W, PER_W)], idx_v)

    ra = (ra0, ra1, ra2, ra3)
    rb = (rb0, rb1, rb2, rb3)
    ga = (ga0, ga1, ga2, ga3)
    gb = (gb0, gb1, gb2, gb3)
    pb = (pb0, pb1, pb2, pb3)
    ob = (ob0, ob1, ob2, ob3)
    sa = (sa0, sa1, sa2, sa3)
    sb = (sb0, sb1, sb2, sb3)
    so = (so0, so1, so2, so3)

    iota = lax.iota(jnp.int32, 16)
    one = lax.broadcast(jnp.int32(1), (16,))

    def splat(v):
        return lax.broadcast(v, (16,))

    def start_gather(c, buf):
        for t in range(CHR // 16):
            v = idx_v[pl.ds(c * CHR + 16 * t, 16)]
            v2 = v + v
            ga[buf][pl.ds(16 * t, 16)] = v2
            gb[buf][pl.ds(16 * t, 16)] = v2 + one
        pltpu.async_copy(table_hbm.at[ga[buf]], ra[buf], sa[buf])
        pltpu.async_copy(table_hbm.at[gb[buf]], rb[buf], sb[buf])
        pltpu.async_copy(peb_hbm.at[pl.ds(c * CHL, CHL)], pb[buf], so[buf])

    def wait_gather(c, buf):
        pltpu.make_async_copy(
            table_hbm.at[ga[buf]], ra[buf], sa[buf]).wait()
        pltpu.make_async_copy(
            table_hbm.at[gb[buf]], rb[buf], sb[buf]).wait()
        pltpu.make_async_copy(
            peb_hbm.at[pl.ds(c * CHL, CHL)], pb[buf], so[buf]).wait()

    def start_write(c, buf):
        pltpu.async_copy(
            ob[buf], out_hbm.at[pl.ds(c * CHL, CHL), :, wid, :, :], so[buf])

    def wait_write(c, buf):
        pltpu.make_async_copy(
            ob[buf], out_hbm.at[pl.ds(c * CHL, CHL), :, wid, :, :], so[buf]
        ).wait()

    def transpose_emit(c, buf):
        for j in range(CHL):
            for p, rp in ((0, ra[buf]), (1, rb[buf])):

                def e_body(e, carry, p=p, rp=rp, j=j):
                    f = p * PW + e
                    ps = pb[buf][j, f, pl.ds(0, 16)]
                    ft = f // FS
                    fs = f % FS
                    for g in range(NG):
                        kr = iota + (j * BPW + 16 * g)
                        cv = plsc.load_gather(rp, [kr, splat(e)])
                        ob[buf][j, ft, fs, pl.ds(16 * g, 16)] = cv + ps
                    return carry

                lax.fori_loop(0, PW, e_body, 0)

    for b in range(NBUF):
        start_gather(b, b)

    def quad_body(q, carry):
        c0 = q * NBUF
        for b in range(NBUF):
            c = c0 + b
            wait_gather(c, b)
            transpose_emit(c, b)
            start_write(c, b)
            nxt = c + NBUF - 1
            prv = (b - 1) % NBUF

            @pl.when(jnp.logical_and(c >= 1, nxt < NCH))
            def _():
                wait_write(c - 1, prv)
                start_gather(nxt, prv)

        return carry

    lax.fori_loop(0, NCH // NBUF, quad_body, 0)

    for b in range(NBUF):
        c = NCH - NBUF + b
        wait_write(c, c % NBUF)


def kernel(x, table, pos_embedding):
    idx_w = (jnp.transpose(x).astype(jnp.int32)
             .reshape(L, NW, BPW).transpose(1, 0, 2).reshape(-1))
    t2 = table.reshape(VOCAB * PARTS, PW)
    pe_b = jnp.broadcast_to(
        pos_embedding[:L, :].astype(jnp.float32)[:, :, None], (L, D, 16))
    raw = _embed_sc(t2, idx_w, pe_b)
    return jnp.transpose(raw, (2, 4, 0, 1, 3)).reshape(B, L, D)


# speedup vs baseline: 1.4884x; 1.1131x over previous
import functools

import jax
import jax.numpy as jnp
from jax import lax
from jax.experimental import pallas as pl
from jax.experimental.pallas import tpu as pltpu
from jax.experimental.pallas import tpu_sc as plsc

VOCAB = 1000000
D = 32
B = 4096
L = 200

NC, NS = 2, 16
NW = NC * NS
BPW = B // NW
NG = BPW // 16
FT, FS = D // 8, 8
CHL = 2
CHR = CHL * BPW
NCH = L // CHL
NBUF = 4
PER_W = L * BPW
PARTS = 4
PW = D // PARTS

_mesh = plsc.VectorSubcoreMesh(core_axis_name="c", subcore_axis_name="s")

_scratch = (
    [pltpu.VMEM((PER_W,), jnp.int32)]
    + [pltpu.VMEM((CHR, PW), jnp.float32)
       for _ in range(PARTS * NBUF)]
    + [pltpu.VMEM((CHR,), jnp.int32)
       for _ in range(PARTS * NBUF)]
    + [pltpu.VMEM((CHL, D, 16), jnp.float32)
       for _ in range(NBUF)]
    + [pltpu.VMEM((CHL, FT, FS, BPW), jnp.float32)
       for _ in range(NBUF)]
    + [pltpu.SemaphoreType.DMA for _ in range(2 * NBUF)]
)


@functools.partial(
    pl.kernel,
    mesh=_mesh,
    out_type=jax.ShapeDtypeStruct((L, FT, NW, FS, BPW), jnp.float32),
    compiler_params=pltpu.CompilerParams(
        use_tc_tiling_on_sc=False, needs_layout_passes=False),
    scratch_types=_scratch,
)
def _embed_sc(table_hbm, idx_hbm, peb_hbm, out_hbm, idx_v, *scr):
    n = PARTS * NBUF
    rows = [[scr[p * NBUF + b] for p in range(PARTS)] for b in range(NBUF)]
    gid = [[scr[n + p * NBUF + b] for p in range(PARTS)] for b in range(NBUF)]
    pb = scr[2 * n:2 * n + NBUF]
    ob = scr[2 * n + NBUF:2 * n + 2 * NBUF]
    sg = scr[2 * n + 2 * NBUF:2 * n + 3 * NBUF]
    so = scr[2 * n + 3 * NBUF:2 * n + 4 * NBUF]

    wid = lax.axis_index("s") * NC + lax.axis_index("c")

    pltpu.sync_copy(idx_hbm.at[pl.ds(wid * PER_W, PER_W)], idx_v)

    iota = lax.iota(jnp.int32, 16)
    pvec = [lax.broadcast(jnp.int32(p), (16,)) for p in range(PARTS)]

    def splat(v):
        return lax.broadcast(v, (16,))

    def start_gather(c, buf):
        for t in range(CHR // 16):
            v = idx_v[pl.ds(c * CHR + 16 * t, 16)]
            vs = v * PARTS
            for p in range(PARTS):
                gid[buf][p][pl.ds(16 * t, 16)] = vs + pvec[p]
        for p in range(PARTS):
            pltpu.async_copy(
                table_hbm.at[gid[buf][p]], rows[buf][p], sg[buf])
        pltpu.async_copy(peb_hbm.at[pl.ds(c * CHL, CHL)], pb[buf], so[buf])

    def wait_gather(c, buf):
        for p in range(PARTS):
            pltpu.make_async_copy(
                table_hbm.at[gid[buf][p]], rows[buf][p], sg[buf]).wait()
        pltpu.make_async_copy(
            peb_hbm.at[pl.ds(c * CHL, CHL)], pb[buf], so[buf]).wait()

    def start_write(c, buf):
        pltpu.async_copy(
            ob[buf], out_hbm.at[pl.ds(c * CHL, CHL), :, wid, :, :], so[buf])

    def wait_write(c, buf):
        pltpu.make_async_copy(
            ob[buf], out_hbm.at[pl.ds(c * CHL, CHL), :, wid, :, :], so[buf]
        ).wait()

    def transpose_emit(c, buf):
        for j in range(CHL):
            for p in range(PARTS):
                rp = rows[buf][p]

                def e_body(e, carry, p=p, rp=rp, j=j):
                    f = p * PW + e
                    ps = pb[buf][j, f, pl.ds(0, 16)]
                    ft = f // FS
                    fs = f % FS
                    for g in range(NG):
                        kr = iota + (j * BPW + 16 * g)
                        cv = plsc.load_gather(rp, [kr, splat(e)])
                        ob[buf][j, ft, fs, pl.ds(16 * g, 16)] = cv + ps
                    return carry

                lax.fori_loop(0, PW, e_body, 0)

    for b in range(NBUF):
        start_gather(b, b)

    def quad_body(q, carry):
        c0 = q * NBUF
        for b in range(NBUF):
            c = c0 + b
            wait_gather(c, b)
            transpose_emit(c, b)
            start_write(c, b)
            nxt = c + NBUF - 1
            prv = (b - 1) % NBUF

            @pl.when(jnp.logical_and(c >= 1, nxt < NCH))
            def _():
                wait_write(c - 1, prv)
                start_gather(nxt, prv)

        return carry

    lax.fori_loop(0, NCH // NBUF, quad_body, 0)

    for b in range(NBUF):
        c = NCH - NBUF + b
        wait_write(c, c % NBUF)


def kernel(x, table, pos_embedding):
    idx_w = (jnp.transpose(x).astype(jnp.int32)
             .reshape(L, NW, BPW).transpose(1, 0, 2).reshape(-1))
    t2 = table.reshape(VOCAB * PARTS, PW)
    pe_b = jnp.broadcast_to(
        pos_embedding[:L, :].astype(jnp.float32)[:, :, None], (L, D, 16))
    raw = _embed_sc(t2, idx_w, pe_b)
    return jnp.transpose(raw, (2, 4, 0, 1, 3)).reshape(B, L, D)
